# baseline (speedup 1.0000x reference)
import functools

import jax
import jax.numpy as jnp
from jax import lax
from jax.experimental import pallas as pl
from jax.experimental.pallas import tpu as pltpu
from jax.experimental.pallas import tpu_sc as plsc

H = 64
NNODES = 50000
HALF = 25088
NPAD = 2 * HALF
TRASH = HALF
SLAB = HALF + 128
NSUB = 16
RPT_W = HALF // NSUB
RPT_Z = SLAB // NSUB
E = 800000
CH = 80
EPT = E // NSUB

_DN = (((1,), (1,)), ((), ()))
_PREC = lax.Precision.HIGHEST


def _seg_mesh():
    return plsc.VectorSubcoreMesh(core_axis_name="c", subcore_axis_name="s")


_SC_PARAMS = pltpu.CompilerParams(use_tc_tiling_on_sc=False)


SB = 2000
NCH_SB = SB // CH
CPT = EPT // CH


def _build_segment(with_deg):
    nb = 4 if with_deg else 5
    main = ((NCH_SB - nb) // nb) * nb
    out_type = [jax.ShapeDtypeStruct((NPAD, H), jnp.float32)]
    scratch = [
        pltpu.VMEM((SB,), jnp.int32),
        pltpu.VMEM((NCH_SB, CH), jnp.int32),
    ]
    scratch += [pltpu.VMEM((CH, H), jnp.float32) for _ in range(nb)]
    scratch += [pltpu.SemaphoreType.DMA((nb,)),
                pltpu.SemaphoreType.DMA((nb,))]
    if with_deg:
        out_type.append(jax.ShapeDtypeStruct((NPAD,), jnp.float32))
        scratch += [pltpu.VMEM((CH,), jnp.float32),
                    pltpu.VMEM((128,), jnp.float32),
                    pltpu.SemaphoreType.DMA]
    scratch.append(pltpu.VMEM_SHARED((SLAB, H), jnp.float32))
    if with_deg:
        scratch.append(pltpu.VMEM_SHARED((SLAB,), jnp.float32))

    def body(*refs):
        if with_deg:
            (x_hbm, src_hbm, dst2_hbm, acc_hbm, deg_hbm,
             src_sb, dloc_sb, r0, r1, r2, r3, gsem, ssem,
             onesv, zdegv, dsem, acc_sh, deg_sh) = refs
            rows = (r0, r1, r2, r3)
        else:
            (x_hbm, src_hbm, dst2_hbm, acc_hbm,
             src_sb, dloc_sb, r0, r1, r2, r3, r4, gsem, ssem,
             acc_sh) = refs
            rows = (r0, r1, r2, r3, r4)
        c = lax.axis_index("c")
        s = lax.axis_index("s")
        cbase = c * HALF

        @pl.loop(0, CH)
        def _(r):
            @pl.loop(0, H, step=16)
            def _(k):
                rows[0][r, pl.ds(k, 16)] = jnp.zeros((16,), jnp.float32)

        zlo = s * RPT_Z
        ztail = RPT_Z % CH
        zmain = RPT_Z - ztail

        @pl.loop(0, zmain, step=CH)
        def _(r):
            pltpu.sync_copy(rows[0], acc_sh.at[pl.ds(zlo + r, CH)])

        pltpu.sync_copy(rows[0].at[pl.ds(0, ztail)],
                        acc_sh.at[pl.ds(zlo + zmain, ztail)])

        if with_deg:
            @pl.loop(0, 128, step=16)
            def _(k):
                zdegv[pl.ds(k, 16)] = jnp.zeros((16,), jnp.float32)

            @pl.loop(0, CH, step=16)
            def _(k):
                onesv[pl.ds(k, 16)] = jnp.ones((16,), jnp.float32)

            dtail = RPT_Z % 128
            dmain = RPT_Z - dtail

            @pl.loop(0, dmain, step=128)
            def _(r):
                pltpu.sync_copy(zdegv, deg_sh.at[pl.ds(zlo + r, 128)])

            pltpu.sync_copy(zdegv.at[pl.ds(0, dtail)],
                            deg_sh.at[pl.ds(zlo + dmain, dtail)])

        plsc.subcore_barrier()

        ebase = s * EPT

        def issue_gather(ch, b):
            pltpu.async_copy(x_hbm.at[src_sb.at[pl.ds(ch * CH, CH)]],
                             rows[b], gsem.at[b])

        def wait_gather(b):
            pltpu.make_async_copy(x_hbm.at[pl.ds(0, CH)], rows[b],
                                  gsem.at[b]).wait()

        def issue_scatter(ch, b):
            pltpu.async_copy(rows[b], acc_sh.at[dloc_sb.at[ch]],
                             ssem.at[b], add=True)

        def wait_scatter(b):
            pltpu.make_async_copy(rows[b], acc_sh.at[pl.ds(0, CH)],
                                  ssem.at[b]).wait()

        @pl.loop(0, CPT, step=NCH_SB)
        def _(rr):
            crow0 = s * CPT + rr
            pltpu.sync_copy(src_hbm.at[pl.ds(crow0 * CH, SB)], src_sb)
            pltpu.sync_copy(dst2_hbm.at[pl.ds(crow0, NCH_SB)], dloc_sb)

            @pl.loop(0, NCH_SB)
            def _(r):
                @pl.loop(0, CH, step=16)
                def _(k):
                    d = dloc_sb[r, pl.ds(k, 16)]
                    loc = d - cbase
                    ok = (loc >= 0) & (loc < HALF)
                    dloc_sb[r, pl.ds(k, 16)] = jnp.where(ok, loc, TRASH)

            if with_deg:
                for ch in range(NCH_SB):
                    pltpu.async_copy(onesv, deg_sh.at[dloc_sb.at[ch]],
                                     dsem, add=True)

            for b in range(nb):
                issue_gather(b, b)

            @pl.loop(0, main, step=nb)
            def _(c0):
                for b in range(nb):
                    ch = c0 + b
                    wait_gather(b)
                    issue_scatter(ch, b)
                    wait_scatter(b)
                    issue_gather(ch + nb, b)

            for ch in range(main, NCH_SB):
                b = ch % nb
                wait_gather(b)
                issue_scatter(ch, b)
                wait_scatter(b)
                if ch + nb < NCH_SB:
                    issue_gather(ch + nb, b)

            if with_deg:
                for ch in range(NCH_SB):
                    pltpu.make_async_copy(onesv, deg_sh.at[pl.ds(0, CH)],
                                          dsem).wait()

        plsc.subcore_barrier()

        lo = s * RPT_W
        g = cbase + lo
        pltpu.sync_copy(acc_sh.at[pl.ds(lo, RPT_W)],
                        acc_hbm.at[pl.ds(g, RPT_W)])
        if with_deg:
            pltpu.sync_copy(deg_sh.at[pl.ds(lo, RPT_W)],
                            deg_hbm.at[pl.ds(g, RPT_W)])

    return pl.kernel(body,
                     out_type=out_type if with_deg else out_type[0],
                     mesh=_seg_mesh(),
                     compiler_params=_SC_PARAMS,
                     scratch_types=scratch)


def _segment_deg(x_pad, src, dst2):
    return _build_segment(True)(x_pad, src, dst2)


def _segment(x_pad, src, dst2):
    return _build_segment(False)(x_pad, src, dst2)


def _lin(x, W, b):
    m, k = x.shape
    rb = 5000

    def body(x_ref, w_ref, b_ref, o_ref):
        o_ref[...] = lax.dot_general(
            x_ref[...], w_ref[...], _DN, precision=_PREC) + b_ref[...]

    return pl.pallas_call(
        body,
        grid=(m // rb,),
        in_specs=[pl.BlockSpec((rb, k), lambda i: (i, 0)),
                  pl.BlockSpec((H, k), lambda i: (0, 0)),
                  pl.BlockSpec((1, H), lambda i: (0, 0))],
        out_specs=pl.BlockSpec((rb, H), lambda i: (i, 0)),
        out_shape=jax.ShapeDtypeStruct((m, H), jnp.float32),
    )(x, W, b.reshape(1, H))


def _encode(xi, xc, xt, Wi, bi, Wc, bc, Wt, bt):
    e_ind = _lin(xi, Wi, bi)
    e_com = _lin(xc, Wc, bc)
    e_tru = _lin(xt, Wt, bt)
    pad = jnp.zeros((NPAD - NNODES, H), jnp.float32)
    return jnp.concatenate([e_ind, e_com, e_tru, pad], axis=0)


RB = NPAD // 8


def _conv_dense(agg, deg1, x, Wl, bl, Wr):
    def body(agg_ref, deg_ref, x_ref, wl_ref, bl_ref, wr_ref, o_ref):
        m = agg_ref[...] / jnp.maximum(deg_ref[...], 1.0)
        y = (lax.dot_general(m, wl_ref[...], _DN, precision=_PREC)
             + bl_ref[...]
             + lax.dot_general(x_ref[...], wr_ref[...], _DN, precision=_PREC))
        o_ref[...] = jnp.maximum(y, 0.0)

    return pl.pallas_call(
        body,
        grid=(NPAD // RB,),
        in_specs=[pl.BlockSpec((RB, H), lambda i: (i, 0)),
                  pl.BlockSpec((RB, 1), lambda i: (i, 0)),
                  pl.BlockSpec((RB, H), lambda i: (i, 0)),
                  pl.BlockSpec((H, H), lambda i: (0, 0)),
                  pl.BlockSpec((1, H), lambda i: (0, 0)),
                  pl.BlockSpec((H, H), lambda i: (0, 0))],
        out_specs=pl.BlockSpec((RB, H), lambda i: (i, 0)),
        out_shape=jax.ShapeDtypeStruct((NPAD, H), jnp.float32),
    )(agg, deg1, x, Wl, bl.reshape(1, H), Wr)


def _conv_dense_cls(agg, deg1, x, Wl, bl, Wr, Wc1, bc1, Wc2, bc2):
    def body(agg_ref, deg_ref, x_ref, wl_ref, bl_ref, wr_ref,
             wc1_ref, bc1_ref, wc2_ref, bc2_ref, o_ref):
        m = agg_ref[...] / jnp.maximum(deg_ref[...], 1.0)
        y = (lax.dot_general(m, wl_ref[...], _DN, precision=_PREC)
             + bl_ref[...]
             + lax.dot_general(x_ref[...], wr_ref[...], _DN, precision=_PREC))
        y = jnp.maximum(y, 0.0)
        h = jnp.maximum(
            lax.dot_general(y, wc1_ref[...], _DN, precision=_PREC)
            + bc1_ref[...], 0.0)
        o_ref[...] = lax.dot_general(
            h, wc2_ref[...], _DN, precision=_PREC) + bc2_ref[...]

    return pl.pallas_call(
        body,
        grid=(NPAD // RB,),
        in_specs=[pl.BlockSpec((RB, H), lambda i: (i, 0)),
                  pl.BlockSpec((RB, 1), lambda i: (i, 0)),
                  pl.BlockSpec((RB, H), lambda i: (i, 0)),
                  pl.BlockSpec((H, H), lambda i: (0, 0)),
                  pl.BlockSpec((1, H), lambda i: (0, 0)),
                  pl.BlockSpec((H, H), lambda i: (0, 0)),
                  pl.BlockSpec((H // 2, H), lambda i: (0, 0)),
                  pl.BlockSpec((1, H // 2), lambda i: (0, 0)),
                  pl.BlockSpec((2, H // 2), lambda i: (0, 0)),
                  pl.BlockSpec((1, 2), lambda i: (0, 0))],
        out_specs=pl.BlockSpec((RB, 2), lambda i: (i, 0)),
        out_shape=jax.ShapeDtypeStruct((NPAD, 2), jnp.float32),
    )(agg, deg1, x, Wl, bl.reshape(1, H), Wr,
      Wc1, bc1.reshape(1, H // 2), Wc2, bc2.reshape(1, 2))


def kernel(x_individual, x_company, x_trust, edge_index,
           W_ind, b_ind, W_com, b_com, W_tru, b_tru,
           W1l, b1l, W1r, W2l, b2l, W2r, Wc1, bc1, Wc2, bc2):
    src = edge_index[0]
    dst2 = edge_index[1].reshape(E // CH, CH)

    x = _encode(x_individual, x_company, x_trust,
                W_ind, b_ind, W_com, b_com, W_tru, b_tru)

    acc1, deg = _segment_deg(x, src, dst2)
    deg1 = deg.reshape(NPAD, 1)
    x1 = _conv_dense(acc1, deg1, x, W1l, b1l, W1r)

    acc2 = _segment(x1, src, dst2)
    out_pad = _conv_dense_cls(acc2, deg1, x1, W2l, b2l, W2r,
                              Wc1, bc1, Wc2, bc2)
    return out_pad[:NNODES]


# speedup vs baseline: 1.0363x; 1.0363x over previous
import functools

import jax
import jax.numpy as jnp
from jax import lax
from jax.experimental import pallas as pl
from jax.experimental.pallas import tpu as pltpu
from jax.experimental.pallas import tpu_sc as plsc

H = 64
NNODES = 50000
HALF = 25088
NPAD = 2 * HALF
TRASH = HALF
SLAB = HALF + 128
NSUB = 16
RPT_W = HALF // NSUB
RPT_Z = SLAB // NSUB
E = 800000
CH = 80
EPT = E // NSUB

_DN = (((1,), (1,)), ((), ()))
_PREC = lax.Precision.HIGHEST


def _seg_mesh():
    return plsc.VectorSubcoreMesh(core_axis_name="c", subcore_axis_name="s")


_SC_PARAMS = pltpu.CompilerParams(use_tc_tiling_on_sc=False)


SB = 2000
NCH_SB = SB // CH
CPT = EPT // CH


def _build_segment(with_deg):
    nb = 3 if with_deg else 4
    main = ((NCH_SB - nb) // nb) * nb
    out_type = [jax.ShapeDtypeStruct((NPAD, H), jnp.float32)]
    scratch = [
        pltpu.VMEM((2, SB), jnp.int32),
        pltpu.VMEM((2, NCH_SB, CH), jnp.int32),
    ]
    scratch += [pltpu.VMEM((CH, H), jnp.float32) for _ in range(nb)]
    scratch += [pltpu.SemaphoreType.DMA((nb,)),
                pltpu.SemaphoreType.DMA((nb,)),
                pltpu.SemaphoreType.DMA]
    if with_deg:
        out_type.append(jax.ShapeDtypeStruct((NPAD,), jnp.float32))
        scratch += [pltpu.VMEM((CH,), jnp.float32),
                    pltpu.SemaphoreType.DMA]
    scratch.append(pltpu.VMEM_SHARED((SLAB, H), jnp.float32))
    if with_deg:
        scratch.append(pltpu.VMEM_SHARED((SLAB,), jnp.float32))

    def body(*refs):
        if with_deg:
            (x_hbm, src_hbm, dst2_hbm, acc_hbm, deg_hbm,
             src_sb, dloc_sb, r0, r1, r2, gsem, ssem, isem,
             onesv, dsem, acc_sh, deg_sh) = refs
            rows = (r0, r1, r2)
        else:
            (x_hbm, src_hbm, dst2_hbm, acc_hbm,
             src_sb, dloc_sb, r0, r1, r2, r3, gsem, ssem, isem,
             acc_sh) = refs
            rows = (r0, r1, r2, r3)
        c = lax.axis_index("c")
        s = lax.axis_index("s")
        cbase = c * HALF

        @pl.loop(0, CH)
        def _(r):
            @pl.loop(0, H, step=16)
            def _(k):
                rows[0][r, pl.ds(k, 16)] = jnp.zeros((16,), jnp.float32)

        zlo = s * RPT_Z
        ztail = RPT_Z % CH
        zmain = RPT_Z - ztail

        @pl.loop(0, zmain, step=CH)
        def _(r):
            pltpu.sync_copy(rows[0], acc_sh.at[pl.ds(zlo + r, CH)])

        pltpu.sync_copy(rows[0].at[pl.ds(0, ztail)],
                        acc_sh.at[pl.ds(zlo + zmain, ztail)])

        if with_deg:
            @pl.loop(0, CH, step=16)
            def _(k):
                onesv[pl.ds(k, 16)] = jnp.ones((16,), jnp.float32)

            dtail = RPT_Z % H
            dmain = RPT_Z - dtail

            @pl.loop(0, dmain, step=H)
            def _(r):
                pltpu.sync_copy(rows[0].at[0], deg_sh.at[pl.ds(zlo + r, H)])

            pltpu.sync_copy(rows[0].at[0, pl.ds(0, dtail)],
                            deg_sh.at[pl.ds(zlo + dmain, dtail)])

        plsc.subcore_barrier()

        crow_base = s * CPT

        def issue_idx(j, p):
            crow0 = crow_base + j * NCH_SB
            pltpu.async_copy(src_hbm.at[pl.ds(crow0 * CH, SB)],
                             src_sb.at[p], isem)
            pltpu.async_copy(dst2_hbm.at[pl.ds(crow0, NCH_SB)],
                             dloc_sb.at[p], isem)

        def wait_idx(p):
            pltpu.make_async_copy(src_hbm.at[pl.ds(0, SB)],
                                  src_sb.at[p], isem).wait()
            pltpu.make_async_copy(dst2_hbm.at[pl.ds(0, NCH_SB)],
                                  dloc_sb.at[p], isem).wait()

        def remap(p):
            @pl.loop(0, NCH_SB)
            def _(r):
                @pl.loop(0, CH, step=16)
                def _(k):
                    d = dloc_sb[p, r, pl.ds(k, 16)]
                    loc = d - cbase
                    ok = (loc >= 0) & (loc < HALF)
                    dloc_sb[p, r, pl.ds(k, 16)] = jnp.where(ok, loc, TRASH)

        def issue_gather(j, ch, b, p):
            crow0 = crow_base + j * NCH_SB
            pltpu.async_copy(
                x_hbm.at[src_sb.at[p, pl.ds(ch * CH, CH)]], rows[b],
                gsem.at[b])

        def wait_gather(b):
            pltpu.make_async_copy(x_hbm.at[pl.ds(0, CH)], rows[b],
                                  gsem.at[b]).wait()

        def issue_scatter(ch, b, p):
            pltpu.async_copy(rows[b], acc_sh.at[dloc_sb.at[p, ch]],
                             ssem.at[b], add=True)
            if with_deg:
                pltpu.async_copy(onesv, deg_sh.at[dloc_sb.at[p, ch]],
                                 dsem, add=True)

        def wait_scatter(b):
            pltpu.make_async_copy(rows[b], acc_sh.at[pl.ds(0, CH)],
                                  ssem.at[b]).wait()

        def ring(j, p, prefetch):
            if prefetch:
                issue_idx(j + 1, 1 - p)

            for b in range(nb):
                issue_gather(j, b, b, p)

            @pl.loop(0, main, step=nb)
            def _(c0):
                for b in range(nb):
                    ch = c0 + b
                    wait_gather(b)
                    issue_scatter(ch, b, p)
                    wait_scatter(b)
                    issue_gather(j, ch + nb, b, p)

            for ch in range(main, NCH_SB):
                b = ch % nb
                wait_gather(b)
                issue_scatter(ch, b, p)
                wait_scatter(b)
                if ch + nb < NCH_SB:
                    issue_gather(j, ch + nb, b, p)

            if with_deg:
                for ch in range(NCH_SB):
                    pltpu.make_async_copy(onesv, deg_sh.at[pl.ds(0, CH)],
                                          dsem).wait()

            if prefetch:
                wait_idx(1 - p)
                remap(1 - p)

        issue_idx(0, 0)
        wait_idx(0)
        remap(0)

        nsb = CPT // NCH_SB

        @pl.loop(0, nsb - 1, step=2)
        def _(j):
            ring(j, 0, True)
            ring(j + 1, 1, True)

        ring(nsb - 1, 0, False)

        plsc.subcore_barrier()

        lo = s * RPT_W
        g = cbase + lo
        pltpu.sync_copy(acc_sh.at[pl.ds(lo, RPT_W)],
                        acc_hbm.at[pl.ds(g, RPT_W)])
        if with_deg:
            pltpu.sync_copy(deg_sh.at[pl.ds(lo, RPT_W)],
                            deg_hbm.at[pl.ds(g, RPT_W)])

    return pl.kernel(body,
                     out_type=out_type if with_deg else out_type[0],
                     mesh=_seg_mesh(),
                     compiler_params=_SC_PARAMS,
                     scratch_types=scratch)


def _segment_deg(x_pad, src, dst2):
    return _build_segment(True)(x_pad, src, dst2)


def _segment(x_pad, src, dst2):
    return _build_segment(False)(x_pad, src, dst2)


def _lin(x, W, b):
    m, k = x.shape
    rb = 5000

    def body(x_ref, w_ref, b_ref, o_ref):
        o_ref[...] = lax.dot_general(
            x_ref[...], w_ref[...], _DN, precision=_PREC) + b_ref[...]

    return pl.pallas_call(
        body,
        grid=(m // rb,),
        in_specs=[pl.BlockSpec((rb, k), lambda i: (i, 0)),
                  pl.BlockSpec((H, k), lambda i: (0, 0)),
                  pl.BlockSpec((1, H), lambda i: (0, 0))],
        out_specs=pl.BlockSpec((rb, H), lambda i: (i, 0)),
        out_shape=jax.ShapeDtypeStruct((m, H), jnp.float32),
    )(x, W, b.reshape(1, H))


def _encode(xi, xc, xt, Wi, bi, Wc, bc, Wt, bt):
    e_ind = _lin(xi, Wi, bi)
    e_com = _lin(xc, Wc, bc)
    e_tru = _lin(xt, Wt, bt)
    pad = jnp.zeros((NPAD - NNODES, H), jnp.float32)
    return jnp.concatenate([e_ind, e_com, e_tru, pad], axis=0)


RB = NPAD // 8


def _conv_dense(agg, deg1, x, Wl, bl, Wr):
    def body(agg_ref, deg_ref, x_ref, wl_ref, bl_ref, wr_ref, o_ref):
        m = agg_ref[...] / jnp.maximum(deg_ref[...], 1.0)
        y = (lax.dot_general(m, wl_ref[...], _DN, precision=_PREC)
             + bl_ref[...]
             + lax.dot_general(x_ref[...], wr_ref[...], _DN, precision=_PREC))
        o_ref[...] = jnp.maximum(y, 0.0)

    return pl.pallas_call(
        body,
        grid=(NPAD // RB,),
        in_specs=[pl.BlockSpec((RB, H), lambda i: (i, 0)),
                  pl.BlockSpec((RB, 1), lambda i: (i, 0)),
                  pl.BlockSpec((RB, H), lambda i: (i, 0)),
                  pl.BlockSpec((H, H), lambda i: (0, 0)),
                  pl.BlockSpec((1, H), lambda i: (0, 0)),
                  pl.BlockSpec((H, H), lambda i: (0, 0))],
        out_specs=pl.BlockSpec((RB, H), lambda i: (i, 0)),
        out_shape=jax.ShapeDtypeStruct((NPAD, H), jnp.float32),
    )(agg, deg1, x, Wl, bl.reshape(1, H), Wr)


def _conv_dense_cls(agg, deg1, x, Wl, bl, Wr, Wc1, bc1, Wc2, bc2):
    def body(agg_ref, deg_ref, x_ref, wl_ref, bl_ref, wr_ref,
             wc1_ref, bc1_ref, wc2_ref, bc2_ref, o_ref):
        m = agg_ref[...] / jnp.maximum(deg_ref[...], 1.0)
        y = (lax.dot_general(m, wl_ref[...], _DN, precision=_PREC)
             + bl_ref[...]
             + lax.dot_general(x_ref[...], wr_ref[...], _DN, precision=_PREC))
        y = jnp.maximum(y, 0.0)
        h = jnp.maximum(
            lax.dot_general(y, wc1_ref[...], _DN, precision=_PREC)
            + bc1_ref[...], 0.0)
        o_ref[...] = lax.dot_general(
            h, wc2_ref[...], _DN, precision=_PREC) + bc2_ref[...]

    return pl.pallas_call(
        body,
        grid=(NPAD // RB,),
        in_specs=[pl.BlockSpec((RB, H), lambda i: (i, 0)),
                  pl.BlockSpec((RB, 1), lambda i: (i, 0)),
                  pl.BlockSpec((RB, H), lambda i: (i, 0)),
                  pl.BlockSpec((H, H), lambda i: (0, 0)),
                  pl.BlockSpec((1, H), lambda i: (0, 0)),
                  pl.BlockSpec((H, H), lambda i: (0, 0)),
                  pl.BlockSpec((H // 2, H), lambda i: (0, 0)),
                  pl.BlockSpec((1, H // 2), lambda i: (0, 0)),
                  pl.BlockSpec((2, H // 2), lambda i: (0, 0)),
                  pl.BlockSpec((1, 2), lambda i: (0, 0))],
        out_specs=pl.BlockSpec((RB, 2), lambda i: (i, 0)),
        out_shape=jax.ShapeDtypeStruct((NPAD, 2), jnp.float32),
    )(agg, deg1, x, Wl, bl.reshape(1, H), Wr,
      Wc1, bc1.reshape(1, H // 2), Wc2, bc2.reshape(1, 2))


def kernel(x_individual, x_company, x_trust, edge_index,
           W_ind, b_ind, W_com, b_com, W_tru, b_tru,
           W1l, b1l, W1r, W2l, b2l, W2r, Wc1, bc1, Wc2, bc2):
    src = edge_index[0]
    dst2 = edge_index[1].reshape(E // CH, CH)

    x = _encode(x_individual, x_company, x_trust,
                W_ind, b_ind, W_com, b_com, W_tru, b_tru)

    acc1, deg = _segment_deg(x, src, dst2)
    deg1 = deg.reshape(NPAD, 1)
    x1 = _conv_dense(acc1, deg1, x, W1l, b1l, W1r)

    acc2 = _segment(x1, src, dst2)
    out_pad = _conv_dense_cls(acc2, deg1, x1, W2l, b2l, W2r,
                              Wc1, bc1, Wc2, bc2)
    return out_pad[:NNODES]
